# baseline (speedup 1.0000x reference)
import functools

import jax
import jax.numpy as jnp
from jax import lax
from jax.experimental import pallas as pl
from jax.experimental.pallas import tpu as pltpu
from jax.experimental.pallas import tpu_sc as plsc

BATCH = 16384
EMBED_DIM = 16
NROW = 1000000

_info = plsc.get_sparse_core_info()
_NC = _info.num_cores
_NS = _info.num_subcores
_NW = _NC * _NS
_BPW = BATCH // _NW
_K = 16
_NCH = _BPW // _K

_mesh = plsc.VectorSubcoreMesh(core_axis_name="c", subcore_axis_name="s")


@functools.partial(
    pl.kernel,
    out_type=jax.ShapeDtypeStruct((BATCH,), jnp.float32),
    mesh=_mesh,
    scratch_types=[
        pltpu.VMEM((_BPW,), jnp.int32),
        pltpu.VMEM((_BPW,), jnp.int32),
        pltpu.VMEM((_K, EMBED_DIM, 128), jnp.float32),
        pltpu.VMEM((_K, EMBED_DIM, 128), jnp.float32),
        pltpu.VMEM((EMBED_DIM, _K), jnp.float32),
        pltpu.VMEM((EMBED_DIM, _K), jnp.float32),
        pltpu.VMEM((4096,), jnp.float32),
        pltpu.VMEM((4096,), jnp.float32),
        pltpu.VMEM((256,), jnp.float32),
        pltpu.VMEM((256,), jnp.float32),
        pltpu.VMEM((16,), jnp.float32),
        pltpu.VMEM((_BPW,), jnp.float32),
        pltpu.SemaphoreType.DMA,
        pltpu.SemaphoreType.DMA,
    ],
    compiler_params=pltpu.CompilerParams(needs_layout_passes=False),
)
def _sc_ncf(uidx_hbm, iidx_hbm, utab, itab, wue_hbm, wie_hbm, b1e_hbm,
            w3e_hbm, b3b_hbm, out_hbm,
            uidx_v, iidx_v, uring, iring, uch_v, ich_v, wu_v, wi_v, b1_v,
            w3_v, b3_v, out_v, usem, isem):
    wid = lax.axis_index("s") * _NC + lax.axis_index("c")
    base = wid * _BPW
    pltpu.sync_copy(uidx_hbm.at[pl.ds(base, _BPW)], uidx_v)
    pltpu.sync_copy(iidx_hbm.at[pl.ds(base, _BPW)], iidx_v)
    pltpu.sync_copy(wue_hbm, wu_v)
    pltpu.sync_copy(wie_hbm, wi_v)
    pltpu.sync_copy(b1e_hbm, b1_v)
    pltpu.sync_copy(w3e_hbm, w3_v)
    pltpu.sync_copy(b3b_hbm, b3_v)

    lane = lax.iota(jnp.int32, 16)

    def fire(kk, q):
        uvec = uidx_v[pl.ds(kk * _K, _K)]
        ivec = iidx_v[pl.ds(kk * _K, _K)]
        ub = pl.multiple_of((uvec[q] >> 7) * 128, 128)
        ib = pl.multiple_of((ivec[q] >> 7) * 128, 128)
        pltpu.async_copy(utab.at[:, pl.ds(ub, 128)], uring.at[q], usem)
        pltpu.async_copy(itab.at[:, pl.ds(ib, 128)], iring.at[q], isem)

    def wait(ring, sem, q):
        pltpu.make_async_copy(utab.at[:, pl.ds(0, 128)], ring.at[q],
                              sem).wait()

    for q in range(_K):
        fire(0, q)

    def body(kk, _):
        uvec = uidx_v[pl.ds(kk * _K, _K)]
        ivec = iidx_v[pl.ds(kk * _K, _K)]
        ucols = uvec & 127
        icols = ivec & 127
        for q in range(_K):
            qv = jnp.full((16,), q, jnp.int32)
            wait(uring, usem, q)
            uv = plsc.load_gather(uring, [qv, lane, lane * 0 + ucols[q]])
            plsc.store_scatter(uch_v, [lane, qv], uv)
            wait(iring, isem, q)
            iv = plsc.load_gather(iring, [qv, lane, lane * 0 + icols[q]])
            plsc.store_scatter(ich_v, [lane, qv], iv)

            @pl.when(kk < _NCH - 1)
            def _():
                fire(kk + 1, q)

        us = [uch_v[d, :] for d in range(EMBED_DIM)]
        vs = [ich_v[d, :] for d in range(EMBED_DIM)]
        o = b3_v[...]
        for k in range(EMBED_DIM):
            acc = b1_v[pl.ds(k * 16, 16)]
            for d in range(EMBED_DIM):
                acc = acc + us[d] * wu_v[pl.ds(k * 256 + d * 16, 16)]
                acc = acc + vs[d] * wi_v[pl.ds(k * 256 + d * 16, 16)]
            o = o + jnp.maximum(acc, 0.0) * w3_v[pl.ds(k * 16, 16)]
        o = 1.0 / (1.0 + jnp.exp(-o))
        out_v[pl.ds(kk * _K, _K)] = o
        return ()

    lax.fori_loop(0, _NCH, body, (), unroll=False)

    pltpu.sync_copy(out_v, out_hbm.at[pl.ds(base, _BPW)])


def kernel(user_indices, item_indices, emb_user, emb_item, W1, b1, W3, b3):
    uidx = user_indices.astype(jnp.int32)
    iidx = item_indices.astype(jnp.int32)
    wue = jnp.broadcast_to(W1[:EMBED_DIM].T[:, :, None],
                           (16, 16, 16)).reshape(-1)
    wie = jnp.broadcast_to(W1[EMBED_DIM:].T[:, :, None],
                           (16, 16, 16)).reshape(-1)
    b1e = jnp.repeat(b1, 16)
    w3e = jnp.repeat(W3[:, 0], 16)
    b3b = jnp.broadcast_to(b3, (16,))
    return _sc_ncf(uidx, iidx, emb_user.T, emb_item.T, wue, wie, b1e, w3e,
                   b3b)


# speedup vs baseline: 1.1872x; 1.1872x over previous
import functools

import jax
import jax.numpy as jnp
from jax import lax
from jax.experimental import pallas as pl
from jax.experimental.pallas import tpu as pltpu
from jax.experimental.pallas import tpu_sc as plsc

BATCH = 16384
EMBED_DIM = 16
NROW = 1000000

_info = plsc.get_sparse_core_info()
_NC = _info.num_cores
_NS = _info.num_subcores
_NW = _NC * _NS
_BPW = BATCH // _NW
_K = 16
_NCH = _BPW // _K

_mesh = plsc.VectorSubcoreMesh(core_axis_name="c", subcore_axis_name="s")


@functools.partial(
    pl.kernel,
    out_type=(
        jax.ShapeDtypeStruct((EMBED_DIM, BATCH), jnp.float32),
        jax.ShapeDtypeStruct((EMBED_DIM, BATCH), jnp.float32),
    ),
    mesh=_mesh,
    scratch_types=[
        pltpu.VMEM((_BPW,), jnp.int32),
        pltpu.VMEM((_BPW,), jnp.int32),
        pltpu.VMEM((_K, EMBED_DIM, 128), jnp.float32),
        pltpu.VMEM((_K, EMBED_DIM, 128), jnp.float32),
        pltpu.VMEM((EMBED_DIM, _BPW), jnp.float32),
        pltpu.VMEM((EMBED_DIM, _BPW), jnp.float32),
        pltpu.SemaphoreType.DMA,
        pltpu.SemaphoreType.DMA,
    ],
    compiler_params=pltpu.CompilerParams(needs_layout_passes=False),
)
def _sc_gather(uidx_hbm, iidx_hbm, utab, itab, uout_hbm, iout_hbm,
               uidx_v, iidx_v, uring, iring, uout_v, iout_v, usem, isem):
    wid = lax.axis_index("s") * _NC + lax.axis_index("c")
    base = wid * _BPW
    pltpu.sync_copy(uidx_hbm.at[pl.ds(base, _BPW)], uidx_v)
    pltpu.sync_copy(iidx_hbm.at[pl.ds(base, _BPW)], iidx_v)

    lane = lax.iota(jnp.int32, 16)
    lane128 = lane * 128

    def fire(kk, q):
        j = kk * _K + q
        uvec = uidx_v[pl.ds(kk * _K, _K)]
        ivec = iidx_v[pl.ds(kk * _K, _K)]
        ub = pl.multiple_of((uvec[q] >> 7) * 128, 128)
        ib = pl.multiple_of((ivec[q] >> 7) * 128, 128)
        pltpu.async_copy(utab.at[:, pl.ds(ub, 128)], uring.at[q], usem)
        pltpu.async_copy(itab.at[:, pl.ds(ib, 128)], iring.at[q], isem)
        return j

    def wait(ring, sem, q):
        pltpu.make_async_copy(utab.at[:, pl.ds(0, 128)], ring.at[q],
                              sem).wait()

    for q in range(_K):
        fire(0, q)

    def body(kk, _):
        uvec = uidx_v[pl.ds(kk * _K, _K)]
        ivec = iidx_v[pl.ds(kk * _K, _K)]
        ucols = uvec & 127
        icols = ivec & 127
        jbase = jnp.full((16,), kk * _K, jnp.int32)
        for q in range(_K):
            jvec = jbase + q
            wait(uring, usem, q)
            uc = jnp.full((16,), 1, jnp.int32) * ucols[q]
            uv = plsc.load_gather(uring, [jnp.full((16,), q, jnp.int32),
                                          lane, uc])
            plsc.store_scatter(uout_v, [lane, jvec], uv)
            wait(iring, isem, q)
            ic = jnp.full((16,), 1, jnp.int32) * icols[q]
            iv = plsc.load_gather(iring, [jnp.full((16,), q, jnp.int32),
                                          lane, ic])
            plsc.store_scatter(iout_v, [lane, jvec], iv)

            @pl.when(kk < _NCH - 1)
            def _():
                fire(kk + 1, q)
        return ()

    lax.fori_loop(0, _NCH, body, (), unroll=False)

    pltpu.sync_copy(uout_v, uout_hbm.at[:, pl.ds(base, _BPW)])
    pltpu.sync_copy(iout_v, iout_hbm.at[:, pl.ds(base, _BPW)])


def _mlp_body(ut_ref, it_ref, w1ut_ref, w1it_ref, b1_ref, w3_ref, b3_ref,
              o_ref):
    h = (jnp.dot(w1ut_ref[...], ut_ref[...],
                 preferred_element_type=jnp.float32)
         + jnp.dot(w1it_ref[...], it_ref[...],
                   preferred_element_type=jnp.float32)
         + b1_ref[...][:, None])
    h = jnp.maximum(h, 0.0)
    o = (jnp.dot(w3_ref[...][None, :], h,
                 preferred_element_type=jnp.float32)[0]
         + b3_ref[...])
    o_ref[...] = jax.nn.sigmoid(o)


def _tc_mlp(ut, it, w1ut, w1it, b1, w3, b3):
    return pl.pallas_call(
        _mlp_body,
        out_shape=jax.ShapeDtypeStruct((BATCH,), jnp.float32),
    )(ut, it, w1ut, w1it, b1, w3, b3)


def kernel(user_indices, item_indices, emb_user, emb_item, W1, b1, W3, b3):
    uidx = user_indices.astype(jnp.int32)
    iidx = item_indices.astype(jnp.int32)
    u_t, i_t = _sc_gather(uidx, iidx, emb_user.T, emb_item.T)
    w1ut = W1[:EMBED_DIM].T
    w1it = W1[EMBED_DIM:].T
    w3 = W3[:, 0]
    return _tc_mlp(u_t, i_t, w1ut, w1it, b1, w3, b3)


# speedup vs baseline: 1.1903x; 1.0027x over previous
import functools

import jax
import jax.numpy as jnp
from jax import lax
from jax.experimental import pallas as pl
from jax.experimental.pallas import tpu as pltpu
from jax.experimental.pallas import tpu_sc as plsc

BATCH = 16384
EMBED_DIM = 16
NROW = 1000000

_info = plsc.get_sparse_core_info()
_NC = _info.num_cores
_NS = _info.num_subcores
_NW = _NC * _NS
_BPW = BATCH // _NW
_K = 16
_NCH = _BPW // _K

_mesh = plsc.VectorSubcoreMesh(core_axis_name="c", subcore_axis_name="s")


@functools.partial(
    pl.kernel,
    out_type=(
        jax.ShapeDtypeStruct((EMBED_DIM, BATCH), jnp.float32),
        jax.ShapeDtypeStruct((EMBED_DIM, BATCH), jnp.float32),
    ),
    mesh=_mesh,
    scratch_types=[
        pltpu.VMEM((_BPW,), jnp.int32),
        pltpu.VMEM((_BPW,), jnp.int32),
        pltpu.VMEM((_K, EMBED_DIM, 128), jnp.float32),
        pltpu.VMEM((_K, EMBED_DIM, 128), jnp.float32),
        pltpu.VMEM((EMBED_DIM, _BPW), jnp.float32),
        pltpu.VMEM((EMBED_DIM, _BPW), jnp.float32),
        pltpu.SemaphoreType.DMA,
        pltpu.SemaphoreType.DMA,
    ],
    compiler_params=pltpu.CompilerParams(needs_layout_passes=False),
)
def _sc_gather(uidx_hbm, iidx_hbm, utab, itab, uout_hbm, iout_hbm,
               uidx_v, iidx_v, uring, iring, uout_v, iout_v, usem, isem):
    wid = lax.axis_index("s") * _NC + lax.axis_index("c")
    base = wid * _BPW
    cu = pltpu.async_copy(uidx_hbm.at[pl.ds(base, _BPW)], uidx_v, usem)
    ci = pltpu.async_copy(iidx_hbm.at[pl.ds(base, _BPW)], iidx_v, isem)
    cu.wait()
    ci.wait()

    lane = lax.iota(jnp.int32, 16)

    def fire(kk, q):
        uvec = uidx_v[pl.ds(kk * _K, _K)]
        ivec = iidx_v[pl.ds(kk * _K, _K)]
        ub = pl.multiple_of((uvec[q] >> 7) * 128, 128)
        ib = pl.multiple_of((ivec[q] >> 7) * 128, 128)
        pltpu.async_copy(utab.at[:, pl.ds(ub, 128)], uring.at[q], usem)
        pltpu.async_copy(itab.at[:, pl.ds(ib, 128)], iring.at[q], isem)

    def wait(ring, sem, q):
        pltpu.make_async_copy(utab.at[:, pl.ds(0, 128)], ring.at[q],
                              sem).wait()

    for q in range(_K):
        fire(0, q)

    def body(kk, _):
        uvec = uidx_v[pl.ds(kk * _K, _K)]
        ivec = iidx_v[pl.ds(kk * _K, _K)]
        ucols = uvec & 127
        icols = ivec & 127
        jbase = jnp.full((16,), kk * _K, jnp.int32)
        for q in range(_K):
            jvec = jbase + q
            wait(uring, usem, q)
            uc = jnp.full((16,), 1, jnp.int32) * ucols[q]
            uv = plsc.load_gather(uring, [jnp.full((16,), q, jnp.int32),
                                          lane, uc])
            plsc.store_scatter(uout_v, [lane, jvec], uv)
            wait(iring, isem, q)
            ic = jnp.full((16,), 1, jnp.int32) * icols[q]
            iv = plsc.load_gather(iring, [jnp.full((16,), q, jnp.int32),
                                          lane, ic])
            plsc.store_scatter(iout_v, [lane, jvec], iv)

            @pl.when(kk < _NCH - 1)
            def _():
                fire(kk + 1, q)
        return ()

    lax.fori_loop(0, _NCH, body, (), unroll=False)

    pltpu.sync_copy(uout_v, uout_hbm.at[:, pl.ds(base, _BPW)])
    pltpu.sync_copy(iout_v, iout_hbm.at[:, pl.ds(base, _BPW)])


def _mlp_body(ut_ref, it_ref, w1ut_ref, w1it_ref, b1_ref, w3_ref, b3_ref,
              o_ref):
    h = (jnp.dot(w1ut_ref[...], ut_ref[...],
                 preferred_element_type=jnp.float32)
         + jnp.dot(w1it_ref[...], it_ref[...],
                   preferred_element_type=jnp.float32)
         + b1_ref[...][:, None])
    h = jnp.maximum(h, 0.0)
    o = (jnp.dot(w3_ref[...][None, :], h,
                 preferred_element_type=jnp.float32)[0]
         + b3_ref[...])
    o_ref[...] = jax.nn.sigmoid(o)


def _tc_mlp(ut, it, w1ut, w1it, b1, w3, b3):
    return pl.pallas_call(
        _mlp_body,
        out_shape=jax.ShapeDtypeStruct((BATCH,), jnp.float32),
    )(ut, it, w1ut, w1it, b1, w3, b3)


def kernel(user_indices, item_indices, emb_user, emb_item, W1, b1, W3, b3):
    uidx = user_indices.astype(jnp.int32)
    iidx = item_indices.astype(jnp.int32)
    u_t, i_t = _sc_gather(uidx, iidx, emb_user.T, emb_item.T)
    w1ut = W1[:EMBED_DIM].T
    w1it = W1[EMBED_DIM:].T
    w3 = W3[:, 0]
    return _tc_mlp(u_t, i_t, w1ut, w1it, b1, w3, b3)
